# Initial kernel scaffold; baseline (speedup 1.0000x reference)
import jax
import jax.numpy as jnp
from jax.experimental import pallas as pl


def kernel(input_ids, attention_mask, emb_table, W, b):
    raise NotImplementedError("write your pallas kernel here")


# speedup vs baseline: 1.6440x; 1.6440x over previous
import functools

import jax
import jax.numpy as jnp
from jax import lax
from jax.experimental import pallas as pl
from jax.experimental.pallas import tpu as pltpu
from jax.experimental.pallas import tpu_sc as plsc


def _lane_reduce_sum(x, lane_iota):
    for sh in (8, 4, 2, 1):
        x = x + x.at[lane_iota ^ sh].get(mode="promise_in_bounds")
    return x


def _make_sc_kernel(B, L, V, E, NC, NS):
    NW = NC * NS
    RPW = B // NW
    LANES = 16
    C0 = 104
    C1 = L - C0
    LPAD = ((L + LANES - 1) // LANES) * LANES
    NCHUNK = LPAD // LANES

    mesh = plsc.VectorSubcoreMesh(core_axis_name="c", subcore_axis_name="s")

    @functools.partial(
        pl.kernel,
        out_type=jax.ShapeDtypeStruct((B,), jnp.float32),
        mesh=mesh,
        compiler_params=pltpu.CompilerParams(use_tc_tiling_on_sc=False),
        scratch_types=[
            pltpu.VMEM((2, C0), jnp.int32),
            pltpu.VMEM((LPAD,), jnp.int32),
            pltpu.VMEM((LPAD, E), jnp.float32),
            pltpu.VMEM((80,), jnp.float32),
            pltpu.VMEM((RPW,), jnp.float32),
            pltpu.SemaphoreType.DMA,
        ],
    )
    def body(ids_hbm, mask_hbm, table_hbm, wb_hbm, out_hbm,
             idx_v, mask_v, rows_v, wb_v, log_v, sem):
        wid = lax.axis_index("s") * NC + lax.axis_index("c")
        base = wid * RPW
        pltpu.sync_copy(wb_hbm, wb_v)
        zero = jnp.zeros((LANES,), jnp.float32)
        mask_v[pl.ds(LPAD - LANES, LANES)] = jnp.zeros((LANES,), jnp.int32)
        for l in range(L, LPAD):
            for j in range(E // LANES):
                rows_v[l, pl.ds(j * LANES, LANES)] = zero

        w0 = wb_v[pl.ds(0, LANES)]
        w1 = wb_v[pl.ds(LANES, LANES)]
        w2 = wb_v[pl.ds(2 * LANES, LANES)]
        w3 = wb_v[pl.ds(3 * LANES, LANES)]
        bias_vec = wb_v[pl.ds(4 * LANES, LANES)]
        lane_iota = lax.iota(jnp.int32, LANES)

        def row_body(r, log_vec):
            tbase = (base + r) * L
            pltpu.sync_copy(ids_hbm.at[pl.ds(tbase, C0)], idx_v.at[0])
            pltpu.sync_copy(ids_hbm.at[pl.ds(tbase + C0, C1)],
                            idx_v.at[1, pl.ds(0, C1)])
            pltpu.sync_copy(mask_hbm.at[pl.ds(tbase, L)],
                            mask_v.at[pl.ds(0, L)])
            cp0 = pltpu.async_copy(table_hbm.at[idx_v.at[0]],
                                   rows_v.at[pl.ds(0, C0)], sem)
            cp1 = pltpu.async_copy(table_hbm.at[idx_v.at[1, pl.ds(0, C1)]],
                                   rows_v.at[pl.ds(C0, C1)], sem)
            cp0.wait()
            cp1.wait()

            def chunk(c, acc):
                cnt, a0, a1, a2, a3 = acc
                mvec = mask_v[pl.ds(c * LANES, LANES)].astype(jnp.float32)
                cnt = cnt + mvec
                lbase = c * LANES
                for j in range(LANES):
                    m = mvec[j]
                    l = lbase + j
                    a0 = a0 + rows_v[l, pl.ds(0, LANES)] * m
                    a1 = a1 + rows_v[l, pl.ds(LANES, LANES)] * m
                    a2 = a2 + rows_v[l, pl.ds(2 * LANES, LANES)] * m
                    a3 = a3 + rows_v[l, pl.ds(3 * LANES, LANES)] * m
                return (cnt, a0, a1, a2, a3)

            cnt, a0, a1, a2, a3 = lax.fori_loop(
                0, NCHUNK, chunk, (zero, zero, zero, zero, zero))

            denom = jnp.maximum(_lane_reduce_sum(cnt, lane_iota), 1e-6)
            inv = 1.0 / denom
            s = (jnp.maximum(a0 * inv, 0.0) * w0
                 + jnp.maximum(a1 * inv, 0.0) * w1
                 + jnp.maximum(a2 * inv, 0.0) * w2
                 + jnp.maximum(a3 * inv, 0.0) * w3)
            logit = _lane_reduce_sum(s + bias_vec, lane_iota)

            lane = lax.rem(r, LANES)
            log_vec = jnp.where(lane_iota == lane, logit, log_vec)

            @pl.when(lane == LANES - 1)
            def _():
                log_v[pl.ds(r - (LANES - 1), LANES)] = log_vec

            return log_vec

        lax.fori_loop(0, RPW, row_body, zero)
        pltpu.sync_copy(log_v, out_hbm.at[pl.ds(base, RPW)])

    return body


def kernel(input_ids, attention_mask, emb_table, W, b):
    B, L = input_ids.shape
    V, E = emb_table.shape
    ids = input_ids.astype(jnp.int32).reshape(-1)
    mask = attention_mask.astype(jnp.int32).reshape(-1)
    wb = jnp.concatenate(
        [W.reshape(-1).astype(jnp.float32),
         b.reshape(-1).astype(jnp.float32),
         jnp.zeros((15,), jnp.float32)])
    info = plsc.get_sparse_core_info()
    sc = _make_sc_kernel(B, L, V, E, info.num_cores, info.num_subcores)
    return sc(ids, mask, emb_table, wb)


# speedup vs baseline: 3.2323x; 1.9662x over previous
import functools

import jax
import jax.numpy as jnp
from jax import lax
from jax.experimental import pallas as pl
from jax.experimental.pallas import tpu as pltpu
from jax.experimental.pallas import tpu_sc as plsc


def _lane_reduce_sum(x, lane_iota):
    for sh in (8, 4, 2, 1):
        x = x + x.at[lane_iota ^ sh].get(mode="promise_in_bounds")
    return x


def _make_sc_kernel(B, L, V, E, NC, NS):
    NW = NC * NS
    RPW = B // NW
    LANES = 16
    C0 = 104
    C1 = L - C0
    LPAD = ((L + LANES - 1) // LANES) * LANES
    NCHUNK = LPAD // LANES
    BLK = 8
    NBLK = RPW // BLK

    mesh = plsc.VectorSubcoreMesh(core_axis_name="c", subcore_axis_name="s")

    @functools.partial(
        pl.kernel,
        out_type=jax.ShapeDtypeStruct((B,), jnp.float32),
        mesh=mesh,
        compiler_params=pltpu.CompilerParams(use_tc_tiling_on_sc=False),
        scratch_types=[
            pltpu.VMEM((2, 2, C0), jnp.int32),
            pltpu.VMEM((2, BLK * L + 8), jnp.int32),
            pltpu.VMEM((2, LPAD, E), jnp.float32),
            pltpu.VMEM((80,), jnp.float32),
            pltpu.VMEM((RPW,), jnp.float32),
            pltpu.SemaphoreType.DMA,
            pltpu.SemaphoreType.DMA,
            pltpu.SemaphoreType.DMA,
            pltpu.SemaphoreType.DMA,
        ],
    )
    def body(ids_hbm, mask_hbm, table_hbm, wb_hbm, out_hbm,
             idx_v, mask_v, rows_v, wb_v, log_v,
             sem_i, sem_m, sem_g0, sem_g1):
        wid = lax.axis_index("s") * NC + lax.axis_index("c")
        base = wid * RPW
        pltpu.sync_copy(wb_hbm, wb_v)
        zero = jnp.zeros((LANES,), jnp.float32)
        for par in range(2):
            for l in range(L, LPAD):
                for j in range(E // LANES):
                    rows_v[par, l, pl.ds(j * LANES, LANES)] = zero

        w0 = wb_v[pl.ds(0, LANES)]
        w1 = wb_v[pl.ds(LANES, LANES)]
        w2 = wb_v[pl.ds(2 * LANES, LANES)]
        w3 = wb_v[pl.ds(3 * LANES, LANES)]
        bias_vec = wb_v[pl.ds(4 * LANES, LANES)]
        lane_iota = lax.iota(jnp.int32, LANES)

        sems_g = (sem_g0, sem_g1)

        def issue_ids(r, slot):
            off = (base + r) * L
            pltpu.async_copy(ids_hbm.at[pl.ds(off, C0)],
                             idx_v.at[slot, 0], sem_i)
            pltpu.async_copy(ids_hbm.at[pl.ds(off + C0, C1)],
                             idx_v.at[slot, 1, pl.ds(0, C1)], sem_i)

        def wait_ids(slot):
            pltpu.make_async_copy(ids_hbm.at[pl.ds(0, C0)],
                                  idx_v.at[slot, 0], sem_i).wait()
            pltpu.make_async_copy(ids_hbm.at[pl.ds(0, C1)],
                                  idx_v.at[slot, 1, pl.ds(0, C1)], sem_i).wait()

        def issue_mask_blk(k, slot):
            off = (base + k * BLK) * L
            pltpu.async_copy(mask_hbm.at[pl.ds(off, BLK * L)],
                             mask_v.at[slot, pl.ds(0, BLK * L)], sem_m)

        def wait_mask_blk(slot):
            pltpu.make_async_copy(mask_hbm.at[pl.ds(0, BLK * L)],
                                  mask_v.at[slot, pl.ds(0, BLK * L)],
                                  sem_m).wait()

        def issue_gather(par):
            sem = sems_g[par]
            pltpu.async_copy(table_hbm.at[idx_v.at[par, 0]],
                             rows_v.at[par, pl.ds(0, C0)], sem)
            pltpu.async_copy(table_hbm.at[idx_v.at[par, 1, pl.ds(0, C1)]],
                             rows_v.at[par, pl.ds(C0, C1)], sem)

        def wait_gather(par):
            sem = sems_g[par]
            pltpu.make_async_copy(table_hbm.at[idx_v.at[par, 0]],
                                  rows_v.at[par, pl.ds(0, C0)], sem).wait()
            pltpu.make_async_copy(table_hbm.at[idx_v.at[par, 1, pl.ds(0, C1)]],
                                  rows_v.at[par, pl.ds(C0, C1)], sem).wait()

        def compute_row(r, par, log_vec):
            blk_slot = lax.rem(lax.div(r, BLK), 2)
            moff = lax.rem(r, BLK) * L

            def chunk(c, acc):
                cnt, a0, a1, a2, a3 = acc
                mvec = mask_v[blk_slot,
                              pl.ds(moff + c * LANES, LANES)].astype(jnp.float32)
                mvec = jnp.where(c * LANES + lane_iota < L, mvec, 0.0)
                cnt = cnt + mvec
                lbase = c * LANES
                for j in range(LANES):
                    m = mvec[j]
                    l = lbase + j
                    a0 = a0 + rows_v[par, l, pl.ds(0, LANES)] * m
                    a1 = a1 + rows_v[par, l, pl.ds(LANES, LANES)] * m
                    a2 = a2 + rows_v[par, l, pl.ds(2 * LANES, LANES)] * m
                    a3 = a3 + rows_v[par, l, pl.ds(3 * LANES, LANES)] * m
                return (cnt, a0, a1, a2, a3)

            cnt, a0, a1, a2, a3 = lax.fori_loop(
                0, NCHUNK, chunk, (zero, zero, zero, zero, zero))

            denom = jnp.maximum(_lane_reduce_sum(cnt, lane_iota), 1e-6)
            inv = 1.0 / denom
            s = (jnp.maximum(a0 * inv, 0.0) * w0
                 + jnp.maximum(a1 * inv, 0.0) * w1
                 + jnp.maximum(a2 * inv, 0.0) * w2
                 + jnp.maximum(a3 * inv, 0.0) * w3)
            logit = _lane_reduce_sum(s + bias_vec, lane_iota)

            lane = lax.rem(r, LANES)
            log_vec = jnp.where(lane_iota == lane, logit, log_vec)

            @pl.when(lane == LANES - 1)
            def _():
                log_v[pl.ds(r - (LANES - 1), LANES)] = log_vec

            return log_vec

        issue_ids(0, 0)
        issue_mask_blk(0, 0)
        wait_ids(0)
        issue_gather(0)
        issue_ids(1, 1)
        wait_mask_blk(0)

        def pair_body(p, log_vec):
            for par in range(2):
                r = 2 * p + par

                @pl.when(r < RPW - 1)
                def _():
                    wait_ids(1 - par)
                    issue_gather(1 - par)

                wait_gather(par)

                @pl.when(r < RPW - 2)
                def _():
                    issue_ids(r + 2, par)

                @pl.when((lax.rem(r, BLK) == 0) & (r < RPW - BLK))
                def _():
                    issue_mask_blk(lax.div(r, BLK) + 1,
                                   lax.rem(lax.div(r, BLK) + 1, 2))

                @pl.when((lax.rem(r, BLK) == BLK - 1) & (r < RPW - 1))
                def _():
                    wait_mask_blk(lax.rem(lax.div(r, BLK) + 1, 2))

                log_vec = compute_row(r, par, log_vec)
            return log_vec

        lax.fori_loop(0, RPW // 2, pair_body, zero)
        pltpu.sync_copy(log_v, out_hbm.at[pl.ds(base, RPW)])

    return body


def kernel(input_ids, attention_mask, emb_table, W, b):
    B, L = input_ids.shape
    V, E = emb_table.shape
    ids = input_ids.astype(jnp.int32).reshape(-1)
    mask = attention_mask.astype(jnp.int32).reshape(-1)
    wb = jnp.concatenate(
        [W.reshape(-1).astype(jnp.float32),
         b.reshape(-1).astype(jnp.float32),
         jnp.zeros((15,), jnp.float32)])
    info = plsc.get_sparse_core_info()
    sc = _make_sc_kernel(B, L, V, E, info.num_cores, info.num_subcores)
    return sc(ids, mask, emb_table, wb)
